# baseline (speedup 1.0000x reference)
import functools

import jax
import jax.numpy as jnp
from jax import lax
from jax.experimental import pallas as pl
from jax.experimental.pallas import tpu as pltpu
from jax.experimental.pallas import tpu_sc as plsc

N_NODES = 10000
N_EDGES = 320000
NUM_COUNTS = 16
NUM_FEATS = 128
LANES = 16

NUM_SEGS = 2
E_SEG = N_EDGES // NUM_SEGS
NUM_CORES = 2
NUM_SUBCORES = 16
SEG_PER_CORE = E_SEG // NUM_CORES
SEG_PER_TILE = SEG_PER_CORE // NUM_SUBCORES
CHUNK = 64
NCH = SEG_PER_TILE // CHUNK
TAIL = SEG_PER_TILE - NCH * CHUNK
N_PAD = 10240
NODES_PER_TILE = N_PAD // NUM_SUBCORES
NP_CHUNK = 64
NP_STEPS = NODES_PER_TILE // NP_CHUNK

TC_BLOCK = 4000
FIN_BLOCK = 2000


def _score_body(cnt_ref, imp_ref, out_ref):
    imp = imp_ref[...]
    m = jnp.max(imp, axis=0, keepdims=True)
    e = jnp.exp(imp - m)
    w = e / jnp.sum(e, axis=0, keepdims=True)
    out_ref[...] = jnp.dot(cnt_ref[...], w, preferred_element_type=jnp.float32)


def _edge_scores(cnt, importance, seg):
    nblk = E_SEG // TC_BLOCK
    return pl.pallas_call(
        _score_body,
        grid=(nblk,),
        in_specs=[
            pl.BlockSpec((TC_BLOCK, NUM_COUNTS), lambda i: (i + seg * nblk, 0)),
            pl.BlockSpec((NUM_COUNTS, NUM_FEATS), lambda i: (0, 0)),
        ],
        out_specs=pl.BlockSpec((TC_BLOCK, NUM_FEATS), lambda i: (i, 0)),
        out_shape=jax.ShapeDtypeStruct((E_SEG, NUM_FEATS), jnp.float32),
    )(cnt, importance)


def _sc_body(seg, emb_ref, src_ref, dst_ref, score_ref, cnt_ref,
             sums_ref, csums_ref,
             acc_e, acc_c,
             idx0, dst0, cnt0, idx1, dst1, cnt1, idx2, dst2, cnt2,
             score0, rows0, score1, rows1,
             idx_t, dst_t,
             sin0, sin1, sin2, sg0, sg1):
    c = lax.axis_index("c")
    s = lax.axis_index("s")
    node0 = s * NODES_PER_TILE
    loc_base = c * SEG_PER_CORE + s * SEG_PER_TILE
    glob_base = seg * E_SEG + loc_base

    idxv = (idx0, idx1, idx2)
    dstv = (dst0, dst1, dst2)
    cntv = (cnt0, cnt1, cnt2)
    sinv = (sin0, sin1, sin2)
    scov = (score0, score1)
    rowv = (rows0, rows1)
    sgv = (sg0, sg1)

    def zero_row(n, carry):
        for b in range(NUM_FEATS // LANES):
            rows0[n, pl.ds(b * LANES, LANES)] = jnp.zeros((LANES,), jnp.float32)
        cnt0[n, :] = jnp.zeros((LANES,), jnp.float32)
        return carry
    lax.fori_loop(0, NP_CHUNK, zero_row, 0)

    def zero_chunk(k, carry):
        r0 = node0 + k * NP_CHUNK
        pltpu.sync_copy(rows0.at[pl.ds(0, NP_CHUNK)], acc_e.at[pl.ds(r0, NP_CHUNK)])
        pltpu.sync_copy(cnt0.at[pl.ds(0, NP_CHUNK)], acc_c.at[pl.ds(r0, NP_CHUNK)])
        return carry
    lax.fori_loop(0, NP_STEPS, zero_chunk, 0)
    plsc.subcore_barrier()

    def in_quad(i, b3, b2):
        gbase = glob_base + i * CHUNK
        lbase = loc_base + i * CHUNK
        return ((src_ref.at[pl.ds(gbase, CHUNK)], idxv[b3]),
                (dst_ref.at[pl.ds(gbase, CHUNK)], dstv[b3]),
                (cnt_ref.at[pl.ds(gbase, CHUNK)], cntv[b3]),
                (score_ref.at[pl.ds(lbase, CHUNK)], scov[b2])), sinv[b3]

    def fire_in(i, b3, b2):
        quads, sem = in_quad(i, b3, b2)
        for src_, dst_ in quads:
            pltpu.async_copy(src_, dst_, sem)

    def wait_in(i, b3, b2):
        quads, sem = in_quad(i, b3, b2)
        for src_, dst_ in quads:
            pltpu.make_async_copy(src_, dst_, sem).wait()

    def fire_g(b3, b2):
        pltpu.async_copy(emb_ref.at[idxv[b3]], rowv[b2], sgv[b2])

    def wait_g(b3, b2):
        pltpu.make_async_copy(emb_ref.at[idxv[b3]], rowv[b2], sgv[b2]).wait()

    def mul(b2, nedges):
        scoreb, rowsb = scov[b2], rowv[b2]

        @plsc.parallel_loop(0, nedges, step=1, unroll=8)
        def _(e):
            for b8 in range(NUM_FEATS // LANES):
                sl = pl.ds(b8 * LANES, LANES)
                rowsb[e, sl] = rowsb[e, sl] * scoreb[e, sl]

    def scatter(b2, b3):
        pltpu.sync_copy(rowv[b2], acc_e.at[dstv[b3]], add=True)
        pltpu.sync_copy(cntv[b3], acc_c.at[dstv[b3]], add=True)

    def turn(k, p):
        b2, b3 = p % 2, p % 3
        nb2, nb3 = (p + 1) % 2, (p + 1) % 3

        @pl.when(k + 1 < NCH)
        def _():
            wait_in(k + 1, nb3, nb2)
            fire_g(nb3, nb2)

        wait_g(b3, b2)
        mul(b2, CHUNK)

        @pl.when(k + 2 < NCH)
        def _():
            fire_in(k + 2, (p + 2) % 3, b2)

        scatter(b2, b3)

    fire_in(0, 0, 0)
    fire_in(1, 1, 1)
    wait_in(0, 0, 0)
    fire_g(0, 0)

    def ring_body(t, carry):
        for p in range(6):
            turn(6 * t + p, p)
        return carry
    lax.fori_loop(0, NCH // 6, ring_body, 0)

    gbase_t = glob_base + NCH * CHUNK
    lbase_t = loc_base + NCH * CHUNK
    pltpu.sync_copy(src_ref.at[pl.ds(gbase_t, TAIL)], idx_t)
    pltpu.sync_copy(dst_ref.at[pl.ds(gbase_t, TAIL)], dst_t)
    pltpu.sync_copy(score_ref.at[pl.ds(lbase_t, TAIL)], score0.at[pl.ds(0, TAIL)])
    pltpu.sync_copy(cnt_ref.at[pl.ds(gbase_t, TAIL)], cnt0.at[pl.ds(0, TAIL)])
    pltpu.async_copy(emb_ref.at[idx_t], rows0.at[pl.ds(0, TAIL)], sg0).wait()
    mul(0, TAIL)
    pltpu.sync_copy(rows0.at[pl.ds(0, TAIL)], acc_e.at[dst_t], add=True)
    pltpu.sync_copy(cnt0.at[pl.ds(0, TAIL)], acc_c.at[dst_t], add=True)
    plsc.subcore_barrier()

    def out_chunk(k, carry):
        r0 = node0 + k * NP_CHUNK
        pltpu.sync_copy(acc_e.at[pl.ds(r0, NP_CHUNK)], rows0.at[pl.ds(0, NP_CHUNK)])
        pltpu.sync_copy(rows0.at[pl.ds(0, NP_CHUNK)], sums_ref.at[c, pl.ds(r0, NP_CHUNK)])
        pltpu.sync_copy(acc_c.at[pl.ds(r0, NP_CHUNK)], cnt0.at[pl.ds(0, NP_CHUNK)])
        pltpu.sync_copy(cnt0.at[pl.ds(0, NP_CHUNK)], csums_ref.at[c, pl.ds(r0, NP_CHUNK)])
        return carry
    lax.fori_loop(0, NP_STEPS, out_chunk, 0)


def _make_sc_aggregate(seg):
    @functools.partial(
        pl.kernel,
        out_type=(
            jax.ShapeDtypeStruct((NUM_CORES, N_PAD, NUM_FEATS), jnp.float32),
            jax.ShapeDtypeStruct((NUM_CORES, N_PAD, NUM_COUNTS), jnp.float32),
        ),
        mesh=plsc.VectorSubcoreMesh(
            core_axis_name="c", subcore_axis_name="s",
            num_cores=NUM_CORES, num_subcores=NUM_SUBCORES),
        scratch_types=[
            pltpu.VMEM_SHARED((N_PAD, NUM_FEATS), jnp.float32),
            pltpu.VMEM_SHARED((N_PAD, NUM_COUNTS), jnp.float32),
            pltpu.VMEM((CHUNK,), jnp.int32),
            pltpu.VMEM((CHUNK,), jnp.int32),
            pltpu.VMEM((CHUNK, NUM_COUNTS), jnp.float32),
            pltpu.VMEM((CHUNK,), jnp.int32),
            pltpu.VMEM((CHUNK,), jnp.int32),
            pltpu.VMEM((CHUNK, NUM_COUNTS), jnp.float32),
            pltpu.VMEM((CHUNK,), jnp.int32),
            pltpu.VMEM((CHUNK,), jnp.int32),
            pltpu.VMEM((CHUNK, NUM_COUNTS), jnp.float32),
            pltpu.VMEM((CHUNK, NUM_FEATS), jnp.float32),
            pltpu.VMEM((CHUNK, NUM_FEATS), jnp.float32),
            pltpu.VMEM((CHUNK, NUM_FEATS), jnp.float32),
            pltpu.VMEM((CHUNK, NUM_FEATS), jnp.float32),
            pltpu.VMEM((TAIL,), jnp.int32),
            pltpu.VMEM((TAIL,), jnp.int32),
            pltpu.SemaphoreType.DMA,
            pltpu.SemaphoreType.DMA,
            pltpu.SemaphoreType.DMA,
            pltpu.SemaphoreType.DMA,
            pltpu.SemaphoreType.DMA,
        ],
        compiler_params=pltpu.CompilerParams(use_tc_tiling_on_sc=False),
    )
    def _sc(emb_ref, src_ref, dst_ref, score_ref, cnt_ref,
            sums_ref, csums_ref, *scratch):
        _sc_body(seg, emb_ref, src_ref, dst_ref, score_ref, cnt_ref,
                 sums_ref, csums_ref, *scratch)
    return _sc


_SC_SEG = tuple(_make_sc_aggregate(seg) for seg in range(NUM_SEGS))


def _final_body(sa_ref, sb_ref, ca_ref, cb_ref, imp_ref, out_ref):
    imp = imp_ref[...]
    m = jnp.max(imp, axis=0, keepdims=True)
    e = jnp.exp(imp - m)
    w = e / jnp.sum(e, axis=0, keepdims=True)
    msg = sa_ref[0] + sa_ref[1] + sb_ref[0] + sb_ref[1]
    csum = ca_ref[0] + ca_ref[1] + cb_ref[0] + cb_ref[1]
    node_score = jnp.dot(csum, w, preferred_element_type=jnp.float32)
    out_ref[...] = msg / node_score


def _finalize(sums0, sums1, csums0, csums1, importance):
    return pl.pallas_call(
        _final_body,
        grid=(N_NODES // FIN_BLOCK,),
        in_specs=[
            pl.BlockSpec((NUM_CORES, FIN_BLOCK, NUM_FEATS), lambda i: (0, i, 0)),
            pl.BlockSpec((NUM_CORES, FIN_BLOCK, NUM_FEATS), lambda i: (0, i, 0)),
            pl.BlockSpec((NUM_CORES, FIN_BLOCK, NUM_COUNTS), lambda i: (0, i, 0)),
            pl.BlockSpec((NUM_CORES, FIN_BLOCK, NUM_COUNTS), lambda i: (0, i, 0)),
            pl.BlockSpec((NUM_COUNTS, NUM_FEATS), lambda i: (0, 0)),
        ],
        out_specs=pl.BlockSpec((FIN_BLOCK, NUM_FEATS), lambda i: (i, 0)),
        out_shape=jax.ShapeDtypeStruct((N_NODES, NUM_FEATS), jnp.float32),
    )(sums0, sums1, csums0, csums1, importance)


def kernel(embedding, edge_index, cnt, importance):
    src = edge_index[0].astype(jnp.int32)
    dst = edge_index[1].astype(jnp.int32)
    score0 = _edge_scores(cnt, importance, 0)
    score1 = _edge_scores(cnt, importance, 1)
    sums0, csums0 = _SC_SEG[0](embedding, src, dst, score0, cnt)
    sums1, csums1 = _SC_SEG[1](embedding, src, dst, score1, cnt)
    return _finalize(sums0, sums1, csums0, csums1, importance)


# speedup vs baseline: 1.0107x; 1.0107x over previous
import functools

import jax
import jax.numpy as jnp
from jax import lax
from jax.experimental import pallas as pl
from jax.experimental.pallas import tpu as pltpu
from jax.experimental.pallas import tpu_sc as plsc

N_NODES = 10000
N_EDGES = 320000
NUM_COUNTS = 16
NUM_FEATS = 128
LANES = 16

NUM_SEGS = 2
E_SEG = N_EDGES // NUM_SEGS
NUM_CORES = 2
NUM_SUBCORES = 16
SEG_PER_CORE = E_SEG // NUM_CORES
SEG_PER_TILE = SEG_PER_CORE // NUM_SUBCORES
CHUNK = 64
NCH = SEG_PER_TILE // CHUNK
TAIL = SEG_PER_TILE - NCH * CHUNK
N_PAD = 10240
NODES_PER_TILE = N_PAD // NUM_SUBCORES
NP_CHUNK = 64
NP_STEPS = NODES_PER_TILE // NP_CHUNK

TC_BLOCK = 4000
FIN_BLOCK = 2000


def _score_body(cnt_ref, imp_ref, out_ref):
    imp = imp_ref[...]
    m = jnp.max(imp, axis=0, keepdims=True)
    e = jnp.exp(imp - m)
    w = e / jnp.sum(e, axis=0, keepdims=True)
    out_ref[...] = jnp.dot(cnt_ref[...], w, preferred_element_type=jnp.float32)


def _edge_scores(cnt, importance, seg):
    nblk = E_SEG // TC_BLOCK
    return pl.pallas_call(
        _score_body,
        grid=(nblk,),
        in_specs=[
            pl.BlockSpec((TC_BLOCK, NUM_COUNTS), lambda i: (i + seg * nblk, 0)),
            pl.BlockSpec((NUM_COUNTS, NUM_FEATS), lambda i: (0, 0)),
        ],
        out_specs=pl.BlockSpec((TC_BLOCK, NUM_FEATS), lambda i: (i, 0)),
        out_shape=jax.ShapeDtypeStruct((E_SEG, NUM_FEATS), jnp.float32),
    )(cnt, importance)


def _sc_body(seg, emb_ref, src_ref, dst_ref, score_ref, cnt_ref,
             sums_ref, csums_ref,
             acc_e, acc_c,
             idx0, dst0, cnt0, idx1, dst1, cnt1, idx2, dst2, cnt2,
             score0, rows0, score1, rows1,
             idx_t, dst_t,
             sin0, sin1, sin2, sg0, sg1):
    c = lax.axis_index("c")
    s = lax.axis_index("s")
    node0 = s * NODES_PER_TILE
    loc_base = c * SEG_PER_CORE + s * SEG_PER_TILE
    glob_base = seg * E_SEG + loc_base

    idxv = (idx0, idx1, idx2)
    dstv = (dst0, dst1, dst2)
    cntv = (cnt0, cnt1, cnt2)
    sinv = (sin0, sin1, sin2)
    scov = (score0, score1)
    rowv = (rows0, rows1)
    sgv = (sg0, sg1)

    def zero_row(n, carry):
        for b in range(NUM_FEATS // LANES):
            rows0[n, pl.ds(b * LANES, LANES)] = jnp.zeros((LANES,), jnp.float32)
        cnt0[n, :] = jnp.zeros((LANES,), jnp.float32)
        return carry
    lax.fori_loop(0, NP_CHUNK, zero_row, 0)

    def zero_chunk(k, carry):
        r0 = node0 + k * NP_CHUNK
        pltpu.sync_copy(rows0.at[pl.ds(0, NP_CHUNK)], acc_e.at[pl.ds(r0, NP_CHUNK)])
        pltpu.sync_copy(cnt0.at[pl.ds(0, NP_CHUNK)], acc_c.at[pl.ds(r0, NP_CHUNK)])
        return carry
    lax.fori_loop(0, NP_STEPS, zero_chunk, 0)
    plsc.subcore_barrier()

    def in_quad(i, b3, b2):
        gbase = glob_base + i * CHUNK
        lbase = loc_base + i * CHUNK
        return ((src_ref.at[pl.ds(gbase, CHUNK)], idxv[b3]),
                (dst_ref.at[pl.ds(gbase, CHUNK)], dstv[b3]),
                (cnt_ref.at[pl.ds(gbase, CHUNK)], cntv[b3]),
                (score_ref.at[pl.ds(lbase, CHUNK)], scov[b2])), sinv[b3]

    def fire_in(i, b3, b2):
        quads, sem = in_quad(i, b3, b2)
        for src_, dst_ in quads:
            pltpu.async_copy(src_, dst_, sem)

    def wait_in(i, b3, b2):
        quads, sem = in_quad(i, b3, b2)
        for src_, dst_ in quads:
            pltpu.make_async_copy(src_, dst_, sem).wait()

    def fire_g(b3, b2):
        pltpu.async_copy(emb_ref.at[idxv[b3]], rowv[b2], sgv[b2])

    def wait_g(b3, b2):
        pltpu.make_async_copy(emb_ref.at[idxv[b3]], rowv[b2], sgv[b2]).wait()

    def mul(b2, nedges):
        scoreb, rowsb = scov[b2], rowv[b2]

        @plsc.parallel_loop(0, nedges, step=1, unroll=4)
        def _(e):
            for b8 in range(NUM_FEATS // LANES):
                sl = pl.ds(b8 * LANES, LANES)
                rowsb[e, sl] = rowsb[e, sl] * scoreb[e, sl]

    def scatter(b2, b3):
        pltpu.sync_copy(rowv[b2], acc_e.at[dstv[b3]], add=True)
        pltpu.sync_copy(cntv[b3], acc_c.at[dstv[b3]], add=True)

    def turn(k, p):
        b2, b3 = p % 2, p % 3
        nb2, nb3 = (p + 1) % 2, (p + 1) % 3

        @pl.when(k + 1 < NCH)
        def _():
            wait_in(k + 1, nb3, nb2)
            fire_g(nb3, nb2)

        wait_g(b3, b2)
        mul(b2, CHUNK)

        @pl.when(k + 2 < NCH)
        def _():
            fire_in(k + 2, (p + 2) % 3, b2)

        scatter(b2, b3)

    fire_in(0, 0, 0)
    fire_in(1, 1, 1)
    wait_in(0, 0, 0)
    fire_g(0, 0)

    def ring_body(t, carry):
        for p in range(6):
            turn(6 * t + p, p)
        return carry
    lax.fori_loop(0, NCH // 6, ring_body, 0)

    gbase_t = glob_base + NCH * CHUNK
    lbase_t = loc_base + NCH * CHUNK
    pltpu.sync_copy(src_ref.at[pl.ds(gbase_t, TAIL)], idx_t)
    pltpu.sync_copy(dst_ref.at[pl.ds(gbase_t, TAIL)], dst_t)
    pltpu.sync_copy(score_ref.at[pl.ds(lbase_t, TAIL)], score0.at[pl.ds(0, TAIL)])
    pltpu.sync_copy(cnt_ref.at[pl.ds(gbase_t, TAIL)], cnt0.at[pl.ds(0, TAIL)])
    pltpu.async_copy(emb_ref.at[idx_t], rows0.at[pl.ds(0, TAIL)], sg0).wait()
    mul(0, TAIL)
    pltpu.sync_copy(rows0.at[pl.ds(0, TAIL)], acc_e.at[dst_t], add=True)
    pltpu.sync_copy(cnt0.at[pl.ds(0, TAIL)], acc_c.at[dst_t], add=True)
    plsc.subcore_barrier()

    def out_chunk(k, carry):
        r0 = node0 + k * NP_CHUNK
        pltpu.sync_copy(acc_e.at[pl.ds(r0, NP_CHUNK)], rows0.at[pl.ds(0, NP_CHUNK)])
        pltpu.sync_copy(rows0.at[pl.ds(0, NP_CHUNK)], sums_ref.at[c, pl.ds(r0, NP_CHUNK)])
        pltpu.sync_copy(acc_c.at[pl.ds(r0, NP_CHUNK)], cnt0.at[pl.ds(0, NP_CHUNK)])
        pltpu.sync_copy(cnt0.at[pl.ds(0, NP_CHUNK)], csums_ref.at[c, pl.ds(r0, NP_CHUNK)])
        return carry
    lax.fori_loop(0, NP_STEPS, out_chunk, 0)


def _make_sc_aggregate(seg):
    @functools.partial(
        pl.kernel,
        out_type=(
            jax.ShapeDtypeStruct((NUM_CORES, N_PAD, NUM_FEATS), jnp.float32),
            jax.ShapeDtypeStruct((NUM_CORES, N_PAD, NUM_COUNTS), jnp.float32),
        ),
        mesh=plsc.VectorSubcoreMesh(
            core_axis_name="c", subcore_axis_name="s",
            num_cores=NUM_CORES, num_subcores=NUM_SUBCORES),
        scratch_types=[
            pltpu.VMEM_SHARED((N_PAD, NUM_FEATS), jnp.float32),
            pltpu.VMEM_SHARED((N_PAD, NUM_COUNTS), jnp.float32),
            pltpu.VMEM((CHUNK,), jnp.int32),
            pltpu.VMEM((CHUNK,), jnp.int32),
            pltpu.VMEM((CHUNK, NUM_COUNTS), jnp.float32),
            pltpu.VMEM((CHUNK,), jnp.int32),
            pltpu.VMEM((CHUNK,), jnp.int32),
            pltpu.VMEM((CHUNK, NUM_COUNTS), jnp.float32),
            pltpu.VMEM((CHUNK,), jnp.int32),
            pltpu.VMEM((CHUNK,), jnp.int32),
            pltpu.VMEM((CHUNK, NUM_COUNTS), jnp.float32),
            pltpu.VMEM((CHUNK, NUM_FEATS), jnp.float32),
            pltpu.VMEM((CHUNK, NUM_FEATS), jnp.float32),
            pltpu.VMEM((CHUNK, NUM_FEATS), jnp.float32),
            pltpu.VMEM((CHUNK, NUM_FEATS), jnp.float32),
            pltpu.VMEM((TAIL,), jnp.int32),
            pltpu.VMEM((TAIL,), jnp.int32),
            pltpu.SemaphoreType.DMA,
            pltpu.SemaphoreType.DMA,
            pltpu.SemaphoreType.DMA,
            pltpu.SemaphoreType.DMA,
            pltpu.SemaphoreType.DMA,
        ],
        compiler_params=pltpu.CompilerParams(use_tc_tiling_on_sc=False),
    )
    def _sc(emb_ref, src_ref, dst_ref, score_ref, cnt_ref,
            sums_ref, csums_ref, *scratch):
        _sc_body(seg, emb_ref, src_ref, dst_ref, score_ref, cnt_ref,
                 sums_ref, csums_ref, *scratch)
    return _sc


_SC_SEG = tuple(_make_sc_aggregate(seg) for seg in range(NUM_SEGS))


def _final_body(sa_ref, sb_ref, ca_ref, cb_ref, imp_ref, out_ref):
    imp = imp_ref[...]
    m = jnp.max(imp, axis=0, keepdims=True)
    e = jnp.exp(imp - m)
    w = e / jnp.sum(e, axis=0, keepdims=True)
    msg = sa_ref[0] + sa_ref[1] + sb_ref[0] + sb_ref[1]
    csum = ca_ref[0] + ca_ref[1] + cb_ref[0] + cb_ref[1]
    node_score = jnp.dot(csum, w, preferred_element_type=jnp.float32)
    out_ref[...] = msg / node_score


def _finalize(sums0, sums1, csums0, csums1, importance):
    return pl.pallas_call(
        _final_body,
        grid=(N_NODES // FIN_BLOCK,),
        in_specs=[
            pl.BlockSpec((NUM_CORES, FIN_BLOCK, NUM_FEATS), lambda i: (0, i, 0)),
            pl.BlockSpec((NUM_CORES, FIN_BLOCK, NUM_FEATS), lambda i: (0, i, 0)),
            pl.BlockSpec((NUM_CORES, FIN_BLOCK, NUM_COUNTS), lambda i: (0, i, 0)),
            pl.BlockSpec((NUM_CORES, FIN_BLOCK, NUM_COUNTS), lambda i: (0, i, 0)),
            pl.BlockSpec((NUM_COUNTS, NUM_FEATS), lambda i: (0, 0)),
        ],
        out_specs=pl.BlockSpec((FIN_BLOCK, NUM_FEATS), lambda i: (i, 0)),
        out_shape=jax.ShapeDtypeStruct((N_NODES, NUM_FEATS), jnp.float32),
    )(sums0, sums1, csums0, csums1, importance)


def kernel(embedding, edge_index, cnt, importance):
    src = edge_index[0].astype(jnp.int32)
    dst = edge_index[1].astype(jnp.int32)
    score0 = _edge_scores(cnt, importance, 0)
    score1 = _edge_scores(cnt, importance, 1)
    sums0, csums0 = _SC_SEG[0](embedding, src, dst, score0, cnt)
    sums1, csums1 = _SC_SEG[1](embedding, src, dst, score1, cnt)
    return _finalize(sums0, sums1, csums0, csums1, importance)
